# Initial kernel scaffold; baseline (speedup 1.0000x reference)
import jax
import jax.numpy as jnp
from jax.experimental import pallas as pl


def kernel(edge_index, num_nodes, num_edges):
    raise NotImplementedError("write your pallas kernel here")


# speedup vs baseline: 32.9742x; 32.9742x over previous
import functools

import jax
import jax.numpy as jnp
from jax import lax
from jax.experimental import pallas as pl
from jax.experimental.pallas import tpu as pltpu
from jax.experimental.pallas import tpu_sc as plsc

OUT_N = 100000
NUM_E = 6400000
NC, NS, L = 2, 16, 16
NW = NC * NS
EPW = NUM_E // NW
CHUNK = 4000
NCHUNK = EPW // CHUNK
NZERO = OUT_N // L


def _sc_body(edge_hbm, partial_hbm, idx_v, counts_v):
    wid = lax.axis_index("s") * NC + lax.axis_index("c")
    base = wid * EPW

    zeros16 = jnp.zeros((L,), jnp.float32)
    ones16 = jnp.ones((L,), jnp.float32)

    def _clear(j, carry):
        counts_v[pl.ds(j * L, L)] = zeros16
        return carry

    lax.fori_loop(0, NZERO, _clear, 0)

    def _chunk(c, carry):
        pltpu.sync_copy(edge_hbm.at[pl.ds(NUM_E + base + c * CHUNK, CHUNK)], idx_v)

        def _vec(i, carry2):
            idx = idx_v[pl.ds(i * L, L)]
            plsc.addupdate_scatter(counts_v, [idx], ones16)
            return carry2

        return lax.fori_loop(0, CHUNK // L, _vec, carry)

    lax.fori_loop(0, NCHUNK, _chunk, 0)

    pltpu.sync_copy(counts_v, partial_hbm.at[wid])


def _tc_reduce_body(parts_ref, out_ref):
    out_ref[...] = jnp.sum(parts_ref[...], axis=0)


def kernel(edge_index, num_nodes, num_edges):
    del num_nodes, num_edges

    mesh = plsc.VectorSubcoreMesh(core_axis_name="c", subcore_axis_name="s")
    partials = pl.kernel(
        _sc_body,
        out_type=jax.ShapeDtypeStruct((NW, OUT_N), jnp.float32),
        mesh=mesh,
        compiler_params=pltpu.CompilerParams(needs_layout_passes=False),
        scratch_types=[
            pltpu.VMEM((CHUNK,), jnp.int32),
            pltpu.VMEM((OUT_N,), jnp.float32),
        ],
    )(edge_index.reshape(2 * NUM_E))

    blk = 4096
    grid = (OUT_N + blk - 1) // blk
    counts = pl.pallas_call(
        _tc_reduce_body,
        out_shape=jax.ShapeDtypeStruct((OUT_N,), jnp.float32),
        grid=(grid,),
        in_specs=[pl.BlockSpec((NW, blk), lambda i: (0, i))],
        out_specs=pl.BlockSpec((blk,), lambda i: (i,)),
    )(partials)
    return counts


# speedup vs baseline: 60.8950x; 1.8467x over previous
import jax
import jax.numpy as jnp
from jax import lax
from jax.experimental import pallas as pl
from jax.experimental.pallas import tpu as pltpu
from jax.experimental.pallas import tpu_sc as plsc

OUT_N = 100000
NUM_E = 6400000
NC, NS, L = 2, 16, 16
NW = NC * NS
CCOL = 2048
NCH = NUM_E // CCOL
KPW = (NCH + NW - 1) // NW
VPC = CCOL // L
UNROLL = 8
ZUNROLL = 10
NZERO = OUT_N // L


def _sc_body(edge_hbm, partial_hbm, buf0, buf1, counts_v, sem0, sem1):
    wid = lax.axis_index("s") * NC + lax.axis_index("c")

    zeros16 = jnp.zeros((L,), jnp.float32)
    ones16 = jnp.ones((L,), jnp.float32)

    def _clear(j, c):
        base = j * (L * ZUNROLL)
        for u in range(ZUNROLL):
            counts_v[pl.ds(base + u * L, L)] = zeros16
        return c

    lax.fori_loop(0, NZERO // ZUNROLL, _clear, 0)

    def _col(k):
        q = jnp.minimum(k * NW + wid, NCH - 1)
        return pl.multiple_of(q * CCOL, CCOL)

    def _start(k, buf, sem):
        pltpu.async_copy(edge_hbm.at[:, pl.ds(_col(k), CCOL)], buf, sem)

    def _wait(buf, sem):
        pltpu.make_async_copy(edge_hbm.at[:, pl.ds(0, CCOL)], buf, sem).wait()

    def _scatter(k, buf):
        live = (k * NW + wid) < NCH
        mask = jnp.full((L,), live)

        def _vec(i, c):
            base = i * (L * UNROLL)
            for u in range(UNROLL):
                idx = buf[1, pl.ds(base + u * L, L)]
                plsc.addupdate_scatter(counts_v, [idx], ones16, mask=mask)
            return c

        lax.fori_loop(0, VPC // UNROLL, _vec, 0)

    _start(0, buf0, sem0)
    _start(1, buf1, sem1)

    def _outer(j, c):
        k0 = 2 * j
        _wait(buf0, sem0)
        _scatter(k0, buf0)

        @pl.when(k0 + 2 < KPW)
        def _():
            _start(k0 + 2, buf0, sem0)

        _wait(buf1, sem1)
        _scatter(k0 + 1, buf1)

        @pl.when(k0 + 3 < KPW)
        def _():
            _start(k0 + 3, buf1, sem1)

        return c

    lax.fori_loop(0, KPW // 2, _outer, 0)

    pltpu.sync_copy(counts_v, partial_hbm.at[wid])


def _tc_reduce_body(parts_ref, out_ref):
    out_ref[...] = jnp.sum(parts_ref[...], axis=0)


def kernel(edge_index, num_nodes, num_edges):
    del num_nodes, num_edges

    mesh = plsc.VectorSubcoreMesh(core_axis_name="c", subcore_axis_name="s")
    partials = pl.kernel(
        _sc_body,
        out_type=jax.ShapeDtypeStruct((NW, OUT_N), jnp.float32),
        mesh=mesh,
        compiler_params=pltpu.CompilerParams(needs_layout_passes=False),
        scratch_types=[
            pltpu.VMEM((2, CCOL), jnp.int32),
            pltpu.VMEM((2, CCOL), jnp.int32),
            pltpu.VMEM((OUT_N,), jnp.float32),
            pltpu.SemaphoreType.DMA,
            pltpu.SemaphoreType.DMA,
        ],
    )(edge_index)

    blk = 4096
    grid = (OUT_N + blk - 1) // blk
    counts = pl.pallas_call(
        _tc_reduce_body,
        out_shape=jax.ShapeDtypeStruct((OUT_N,), jnp.float32),
        grid=(grid,),
        in_specs=[pl.BlockSpec((NW, blk), lambda i: (0, i))],
        out_specs=pl.BlockSpec((blk,), lambda i: (i,)),
    )(partials)
    return counts


# speedup vs baseline: 61.3998x; 1.0083x over previous
import jax
import jax.numpy as jnp
from jax import lax
from jax.experimental import pallas as pl
from jax.experimental.pallas import tpu as pltpu
from jax.experimental.pallas import tpu_sc as plsc

OUT_N = 100000
NUM_E = 6400000
NC, NS, L = 2, 16, 16
NW = NC * NS
CCOL = 2560
NCH = NUM_E // CCOL
KPW = 2 * ((NCH // NW + 2) // 2)
VPC = CCOL // L
UNROLL = 16
ZUNROLL = 10
NZERO = OUT_N // L


def _sc_body(edge_hbm, partial_hbm, buf0, buf1, counts_v, sem0, sem1):
    wid = lax.axis_index("s") * NC + lax.axis_index("c")

    zeros16 = jnp.zeros((L,), jnp.float32)
    ones16 = jnp.ones((L,), jnp.float32)

    def _col(k):
        q = jnp.minimum(k * NW + wid, NCH - 1)
        return pl.multiple_of(q * CCOL, CCOL)

    def _start(k, buf, sem):
        pltpu.async_copy(edge_hbm.at[:, pl.ds(_col(k), CCOL)], buf, sem)

    _start(0, buf0, sem0)
    _start(1, buf1, sem1)

    def _clear(j, c):
        base = j * (L * ZUNROLL)
        for u in range(ZUNROLL):
            counts_v[pl.ds(base + u * L, L)] = zeros16
        return c

    lax.fori_loop(0, NZERO // ZUNROLL, _clear, 0)

    def _wait(buf, sem):
        pltpu.make_async_copy(edge_hbm.at[:, pl.ds(0, CCOL)], buf, sem).wait()

    def _scatter(k, buf):
        live = (k * NW + wid) < NCH
        mask = jnp.full((L,), live)

        def _vec(i, c):
            base = i * (L * UNROLL)
            for u in range(UNROLL):
                idx = buf[1, pl.ds(base + u * L, L)]
                plsc.addupdate_scatter(counts_v, [idx], ones16, mask=mask)
            return c

        lax.fori_loop(0, VPC // UNROLL, _vec, 0)

    def _outer(j, c):
        k0 = 2 * j
        _wait(buf0, sem0)
        _scatter(k0, buf0)

        @pl.when(k0 + 2 < KPW)
        def _():
            _start(k0 + 2, buf0, sem0)

        _wait(buf1, sem1)
        _scatter(k0 + 1, buf1)

        @pl.when(k0 + 3 < KPW)
        def _():
            _start(k0 + 3, buf1, sem1)

        return c

    lax.fori_loop(0, KPW // 2, _outer, 0)

    pltpu.sync_copy(counts_v, partial_hbm.at[wid])


def _tc_reduce_body(parts_ref, out_ref):
    out_ref[...] = jnp.sum(parts_ref[...], axis=0)


def kernel(edge_index, num_nodes, num_edges):
    del num_nodes, num_edges

    mesh = plsc.VectorSubcoreMesh(core_axis_name="c", subcore_axis_name="s")
    partials = pl.kernel(
        _sc_body,
        out_type=jax.ShapeDtypeStruct((NW, OUT_N), jnp.float32),
        mesh=mesh,
        compiler_params=pltpu.CompilerParams(needs_layout_passes=False),
        scratch_types=[
            pltpu.VMEM((2, CCOL), jnp.int32),
            pltpu.VMEM((2, CCOL), jnp.int32),
            pltpu.VMEM((OUT_N,), jnp.float32),
            pltpu.SemaphoreType.DMA,
            pltpu.SemaphoreType.DMA,
        ],
    )(edge_index)

    blk = 4096
    grid = (OUT_N + blk - 1) // blk
    counts = pl.pallas_call(
        _tc_reduce_body,
        out_shape=jax.ShapeDtypeStruct((OUT_N,), jnp.float32),
        grid=(grid,),
        in_specs=[pl.BlockSpec((NW, blk), lambda i: (0, i))],
        out_specs=pl.BlockSpec((blk,), lambda i: (i,)),
    )(partials)
    return counts
